# baseline (speedup 1.0000x reference)
import jax
import jax.numpy as jnp
from jax import lax
from jax.experimental import pallas as pl
from jax.experimental.pallas import tpu as pltpu
from jax.experimental.pallas import tpu_sc as plsc

N_RAYS = 1048576
NUM_CAMS = 1000
CAM_PAD = 1024
NC = 2
NS = 16
L = 16
NW = NC * NS
RAYS_PER_W = N_RAYS // NW
CHUNK = 2048
NCHUNKS = RAYS_PER_W // CHUNK
GROUPS = CHUNK // L


def _full(v):
    return jnp.full((L,), v, dtype=jnp.int32)


def _body(rays_o_hbm, rays_d_hbm, ids_hbm, rvec_hbm, tvec_hbm,
          out_o_hbm, out_d_hbm,
          rvec_v, tvec_v, tbl_v, o_v, d_v, ids_v, oo_v, dd_v):
    wid = lax.axis_index("s") * NC + lax.axis_index("c")
    iota = lax.iota(jnp.int32, L)

    pltpu.sync_copy(rvec_hbm, rvec_v.at[pl.ds(0, NUM_CAMS)])
    pltpu.sync_copy(tvec_hbm, tvec_v.at[pl.ds(0, NUM_CAMS)])

    def build(g, carry):
        base = g * L
        cam = base + iota
        x = plsc.load_gather(rvec_v, [cam, _full(0)])
        y = plsc.load_gather(rvec_v, [cam, _full(1)])
        z = plsc.load_gather(rvec_v, [cam, _full(2)])
        w = plsc.load_gather(rvec_v, [cam, _full(3)])
        t0 = plsc.load_gather(tvec_v, [cam, _full(0)])
        t1 = plsc.load_gather(tvec_v, [cam, _full(1)])
        t2 = plsc.load_gather(tvec_v, [cam, _full(2)])
        theta2 = 1e-5 + x * x + y * y + z * z + w * w
        a = 2.0 / theta2
        axx = a * x * x
        ayy = a * y * y
        azz = a * z * z
        axy = a * x * y
        axz = a * x * z
        ayz = a * y * z
        axw = a * x * w
        ayw = a * y * w
        azw = a * z * w
        tbl_v[0, pl.ds(base, L)] = 1.0 - ayy - azz
        tbl_v[1, pl.ds(base, L)] = axy - azw
        tbl_v[2, pl.ds(base, L)] = axz + ayw
        tbl_v[3, pl.ds(base, L)] = axy + azw
        tbl_v[4, pl.ds(base, L)] = 1.0 - axx - azz
        tbl_v[5, pl.ds(base, L)] = ayz - axw
        tbl_v[6, pl.ds(base, L)] = axz - ayw
        tbl_v[7, pl.ds(base, L)] = ayz + axw
        tbl_v[8, pl.ds(base, L)] = 1.0 - axx - ayy
        tbl_v[9, pl.ds(base, L)] = t0
        tbl_v[10, pl.ds(base, L)] = t1
        tbl_v[11, pl.ds(base, L)] = t2
        return carry

    lax.fori_loop(0, CAM_PAD // L, build, 0)

    wbase = wid * RAYS_PER_W

    def chunk_body(c, carry):
        base = wbase + c * CHUNK
        pltpu.sync_copy(rays_o_hbm.at[pl.ds(base, CHUNK)], o_v)
        pltpu.sync_copy(rays_d_hbm.at[pl.ds(base, CHUNK)], d_v)
        pltpu.sync_copy(ids_hbm.at[pl.ds(base, CHUNK)], ids_v)

        def group(g, carry2):
            r = g * L
            ridx = r + iota
            ids16 = ids_v[pl.ds(r, L)]
            o0 = plsc.load_gather(o_v, [ridx, _full(0)])
            o1 = plsc.load_gather(o_v, [ridx, _full(1)])
            o2 = plsc.load_gather(o_v, [ridx, _full(2)])
            d0 = plsc.load_gather(d_v, [ridx, _full(0)])
            d1 = plsc.load_gather(d_v, [ridx, _full(1)])
            d2 = plsc.load_gather(d_v, [ridx, _full(2)])
            c00 = plsc.load_gather(tbl_v, [_full(0), ids16])
            c01 = plsc.load_gather(tbl_v, [_full(1), ids16])
            c02 = plsc.load_gather(tbl_v, [_full(2), ids16])
            c10 = plsc.load_gather(tbl_v, [_full(3), ids16])
            c11 = plsc.load_gather(tbl_v, [_full(4), ids16])
            c12 = plsc.load_gather(tbl_v, [_full(5), ids16])
            c20 = plsc.load_gather(tbl_v, [_full(6), ids16])
            c21 = plsc.load_gather(tbl_v, [_full(7), ids16])
            c22 = plsc.load_gather(tbl_v, [_full(8), ids16])
            t0 = plsc.load_gather(tbl_v, [_full(9), ids16])
            t1 = plsc.load_gather(tbl_v, [_full(10), ids16])
            t2 = plsc.load_gather(tbl_v, [_full(11), ids16])
            plsc.store_scatter(oo_v, [ridx, _full(0)], o0 + t0)
            plsc.store_scatter(oo_v, [ridx, _full(1)], o1 + t1)
            plsc.store_scatter(oo_v, [ridx, _full(2)], o2 + t2)
            plsc.store_scatter(dd_v, [ridx, _full(0)],
                               d0 * c00 + d1 * c01 + d2 * c02)
            plsc.store_scatter(dd_v, [ridx, _full(1)],
                               d0 * c10 + d1 * c11 + d2 * c12)
            plsc.store_scatter(dd_v, [ridx, _full(2)],
                               d0 * c20 + d1 * c21 + d2 * c22)
            return carry2

        lax.fori_loop(0, GROUPS, group, 0)
        pltpu.sync_copy(oo_v, out_o_hbm.at[pl.ds(base, CHUNK)])
        pltpu.sync_copy(dd_v, out_d_hbm.at[pl.ds(base, CHUNK)])
        return carry

    lax.fori_loop(0, NCHUNKS, chunk_body, 0)


_sc_kernel = pl.kernel(
    _body,
    out_type=(jax.ShapeDtypeStruct((N_RAYS, 3), jnp.float32),
              jax.ShapeDtypeStruct((N_RAYS, 3), jnp.float32)),
    mesh=plsc.VectorSubcoreMesh(core_axis_name="c", subcore_axis_name="s"),
    compiler_params=pltpu.CompilerParams(needs_layout_passes=False,
                                         use_tc_tiling_on_sc=False),
    scratch_types=[
        pltpu.VMEM((CAM_PAD, 4), jnp.float32),
        pltpu.VMEM((CAM_PAD, 3), jnp.float32),
        pltpu.VMEM((12, CAM_PAD), jnp.float32),
        pltpu.VMEM((CHUNK, 4), jnp.float32),
        pltpu.VMEM((CHUNK, 4), jnp.float32),
        pltpu.VMEM((CHUNK,), jnp.int32),
        pltpu.VMEM((CHUNK, 3), jnp.float32),
        pltpu.VMEM((CHUNK, 3), jnp.float32),
    ],
)


def kernel(rays_o, rays_d, rays_id, rvec, tvec):
    ids = rays_id.astype(jnp.int32)
    return _sc_kernel(rays_o, rays_d, ids, rvec, tvec)


# speedup vs baseline: 43.5743x; 43.5743x over previous
import jax
import jax.numpy as jnp
from jax import lax
from jax.experimental import pallas as pl
from jax.experimental.pallas import tpu as pltpu
from jax.experimental.pallas import tpu_sc as plsc

N_RAYS = 1048576
NUM_CAMS = 1000
CAM_PAD = 1024
NC = 2
NS = 16
L = 16
NW = NC * NS
RAYS_PER_W = N_RAYS // NW
BLK = 128
NBLK = N_RAYS // BLK
CHUNK = 2048
CBLK = CHUNK // BLK
NCHUNKS = RAYS_PER_W // CHUNK


def _full(v):
    return jnp.full((L,), v, dtype=jnp.int32)


def _body(rays_o_hbm, rays_d_hbm, ids_hbm, rvec_hbm, tvec_hbm,
          out_o_hbm, out_d_hbm,
          rvec_v, tvec_v, tbl_v,
          o_v0, o_v1, d_v0, d_v1, ids_v0, ids_v1,
          oo_v0, oo_v1, dd_v0, dd_v1,
          sem_in0, sem_in1, sem_out0, sem_out1):
    wid = lax.axis_index("s") * NC + lax.axis_index("c")
    iota = lax.iota(jnp.int32, L)
    o_v = (o_v0, o_v1)
    d_v = (d_v0, d_v1)
    ids_v = (ids_v0, ids_v1)
    oo_v = (oo_v0, oo_v1)
    dd_v = (dd_v0, dd_v1)
    sem_in = (sem_in0, sem_in1)
    sem_out = (sem_out0, sem_out1)

    pltpu.sync_copy(rvec_hbm, rvec_v.at[pl.ds(0, NUM_CAMS)])
    pltpu.sync_copy(tvec_hbm, tvec_v.at[pl.ds(0, NUM_CAMS)])

    def build(g, carry):
        base = g * L
        cam = base + iota
        x = plsc.load_gather(rvec_v, [cam, _full(0)])
        y = plsc.load_gather(rvec_v, [cam, _full(1)])
        z = plsc.load_gather(rvec_v, [cam, _full(2)])
        w = plsc.load_gather(rvec_v, [cam, _full(3)])
        t0 = plsc.load_gather(tvec_v, [cam, _full(0)])
        t1 = plsc.load_gather(tvec_v, [cam, _full(1)])
        t2 = plsc.load_gather(tvec_v, [cam, _full(2)])
        theta2 = 1e-5 + x * x + y * y + z * z + w * w
        a = 2.0 / theta2
        axx = a * x * x
        ayy = a * y * y
        azz = a * z * z
        axy = a * x * y
        axz = a * x * z
        ayz = a * y * z
        axw = a * x * w
        ayw = a * y * w
        azw = a * z * w
        tbl_v[0, pl.ds(base, L)] = 1.0 - ayy - azz
        tbl_v[1, pl.ds(base, L)] = axy - azw
        tbl_v[2, pl.ds(base, L)] = axz + ayw
        tbl_v[3, pl.ds(base, L)] = axy + azw
        tbl_v[4, pl.ds(base, L)] = 1.0 - axx - azz
        tbl_v[5, pl.ds(base, L)] = ayz - axw
        tbl_v[6, pl.ds(base, L)] = axz - ayw
        tbl_v[7, pl.ds(base, L)] = ayz + axw
        tbl_v[8, pl.ds(base, L)] = 1.0 - axx - ayy
        tbl_v[9, pl.ds(base, L)] = t0
        tbl_v[10, pl.ds(base, L)] = t1
        tbl_v[11, pl.ds(base, L)] = t2
        return carry

    lax.fori_loop(0, CAM_PAD // L, build, 0)

    wbase = wid * RAYS_PER_W
    zeros = jnp.zeros((L,), jnp.float32)

    def in_copies(c):
        p = c % 2
        base = wbase + c * CHUNK
        bblk = base // BLK
        return (
            pltpu.make_async_copy(rays_o_hbm.at[pl.ds(bblk, CBLK)],
                                  o_v[p], sem_in[p]),
            pltpu.make_async_copy(rays_d_hbm.at[pl.ds(bblk, CBLK)],
                                  d_v[p], sem_in[p]),
            pltpu.make_async_copy(ids_hbm.at[pl.ds(base, CHUNK)],
                                  ids_v[p], sem_in[p]),
        )

    def out_copies(c):
        p = c % 2
        bblk = (wbase + c * CHUNK) // BLK
        return (
            pltpu.make_async_copy(oo_v[p], out_o_hbm.at[pl.ds(bblk, CBLK)],
                                  sem_out[p]),
            pltpu.make_async_copy(dd_v[p], out_d_hbm.at[pl.ds(bblk, CBLK)],
                                  sem_out[p]),
        )

    def compute_chunk(c):
        p = c % 2
        o_v_, d_v_, ids_v_, oo_v_, dd_v_ = (
            o_v[p], d_v[p], ids_v[p], oo_v[p], dd_v[p])

        def block_body(b, carry2):
            for j in range(BLK // L):
                l0 = j * L
                ids16 = ids_v_[pl.ds(b * BLK + l0, L)]
                o0 = o_v_[b, 0, pl.ds(l0, L)]
                o1 = o_v_[b, 1, pl.ds(l0, L)]
                o2 = o_v_[b, 2, pl.ds(l0, L)]
                d0 = d_v_[b, 0, pl.ds(l0, L)]
                d1 = d_v_[b, 1, pl.ds(l0, L)]
                d2 = d_v_[b, 2, pl.ds(l0, L)]
                c00 = plsc.load_gather(tbl_v, [_full(0), ids16])
                c01 = plsc.load_gather(tbl_v, [_full(1), ids16])
                c02 = plsc.load_gather(tbl_v, [_full(2), ids16])
                c10 = plsc.load_gather(tbl_v, [_full(3), ids16])
                c11 = plsc.load_gather(tbl_v, [_full(4), ids16])
                c12 = plsc.load_gather(tbl_v, [_full(5), ids16])
                c20 = plsc.load_gather(tbl_v, [_full(6), ids16])
                c21 = plsc.load_gather(tbl_v, [_full(7), ids16])
                c22 = plsc.load_gather(tbl_v, [_full(8), ids16])
                t0 = plsc.load_gather(tbl_v, [_full(9), ids16])
                t1 = plsc.load_gather(tbl_v, [_full(10), ids16])
                t2 = plsc.load_gather(tbl_v, [_full(11), ids16])
                oo_v_[b, 0, pl.ds(l0, L)] = o0 + t0
                oo_v_[b, 1, pl.ds(l0, L)] = o1 + t1
                oo_v_[b, 2, pl.ds(l0, L)] = o2 + t2
                oo_v_[b, 3, pl.ds(l0, L)] = zeros
                dd_v_[b, 0, pl.ds(l0, L)] = d0 * c00 + d1 * c01 + d2 * c02
                dd_v_[b, 1, pl.ds(l0, L)] = d0 * c10 + d1 * c11 + d2 * c12
                dd_v_[b, 2, pl.ds(l0, L)] = d0 * c20 + d1 * c21 + d2 * c22
                dd_v_[b, 3, pl.ds(l0, L)] = zeros
            return carry2

        lax.fori_loop(0, CBLK, block_body, 0)

    for cp in in_copies(0):
        cp.start()
    for c in range(NCHUNKS):
        if c + 1 < NCHUNKS:
            for cp in in_copies(c + 1):
                cp.start()
        for cp in in_copies(c):
            cp.wait()
        if c >= 2:
            for cp in out_copies(c - 2):
                cp.wait()
        compute_chunk(c)
        for cp in out_copies(c):
            cp.start()
    for cp in out_copies(NCHUNKS - 2):
        cp.wait()
    for cp in out_copies(NCHUNKS - 1):
        cp.wait()


_sc_kernel = pl.kernel(
    _body,
    out_type=(jax.ShapeDtypeStruct((NBLK, 4, BLK), jnp.float32),
              jax.ShapeDtypeStruct((NBLK, 4, BLK), jnp.float32)),
    mesh=plsc.VectorSubcoreMesh(core_axis_name="c", subcore_axis_name="s"),
    compiler_params=pltpu.CompilerParams(needs_layout_passes=False,
                                         use_tc_tiling_on_sc=False),
    scratch_types=[
        pltpu.VMEM((CAM_PAD, 4), jnp.float32),
        pltpu.VMEM((CAM_PAD, 3), jnp.float32),
        pltpu.VMEM((12, CAM_PAD), jnp.float32),
        pltpu.VMEM((CBLK, 4, BLK), jnp.float32),
        pltpu.VMEM((CBLK, 4, BLK), jnp.float32),
        pltpu.VMEM((CBLK, 4, BLK), jnp.float32),
        pltpu.VMEM((CBLK, 4, BLK), jnp.float32),
        pltpu.VMEM((CHUNK,), jnp.int32),
        pltpu.VMEM((CHUNK,), jnp.int32),
        pltpu.VMEM((CBLK, 4, BLK), jnp.float32),
        pltpu.VMEM((CBLK, 4, BLK), jnp.float32),
        pltpu.VMEM((CBLK, 4, BLK), jnp.float32),
        pltpu.VMEM((CBLK, 4, BLK), jnp.float32),
        pltpu.SemaphoreType.DMA,
        pltpu.SemaphoreType.DMA,
        pltpu.SemaphoreType.DMA,
        pltpu.SemaphoreType.DMA,
    ],
)


def kernel(rays_o, rays_d, rays_id, rvec, tvec):
    ids = rays_id.astype(jnp.int32)
    o3 = rays_o.reshape(NBLK, BLK, 4).transpose(0, 2, 1)
    d3 = rays_d.reshape(NBLK, BLK, 4).transpose(0, 2, 1)
    oo3, dd3 = _sc_kernel(o3, d3, ids, rvec, tvec)
    out_o = oo3.transpose(0, 2, 1).reshape(N_RAYS, 4)[:, :3]
    out_d = dd3.transpose(0, 2, 1).reshape(N_RAYS, 4)[:, :3]
    return (out_o, out_d)


# speedup vs baseline: 51.3888x; 1.1793x over previous
import jax
import jax.numpy as jnp
from jax import lax
from jax.experimental import pallas as pl
from jax.experimental.pallas import tpu as pltpu
from jax.experimental.pallas import tpu_sc as plsc

N_RAYS = 1048576
NUM_CAMS = 1000
CAM_PAD = 1024
NC = 2
NS = 16
L = 16
NW = NC * NS
RAYS_PER_W = N_RAYS // NW
BLK = 128
NBLK = N_RAYS // BLK
CHUNK = 2048
CBLK = CHUNK // BLK
NCHUNKS = RAYS_PER_W // CHUNK


def _full(v):
    return jnp.full((L,), v, dtype=jnp.int32)


def _body(rays_o_hbm, rays_d_hbm, ids_hbm, rvec_hbm, tvec_hbm,
          out_o_hbm, out_d_hbm,
          rvec_v, tvec_v, tbl_v,
          o_v0, o_v1, d_v0, d_v1, ids_v0, ids_v1,
          oo_v0, oo_v1, dd_v0, dd_v1,
          sem_in0, sem_in1, sem_out0, sem_out1):
    wid = lax.axis_index("s") * NC + lax.axis_index("c")
    iota = lax.iota(jnp.int32, L)
    o_v = (o_v0, o_v1)
    d_v = (d_v0, d_v1)
    ids_v = (ids_v0, ids_v1)
    oo_v = (oo_v0, oo_v1)
    dd_v = (dd_v0, dd_v1)
    sem_in = (sem_in0, sem_in1)
    sem_out = (sem_out0, sem_out1)

    pltpu.sync_copy(rvec_hbm, rvec_v.at[pl.ds(0, NUM_CAMS)])
    pltpu.sync_copy(tvec_hbm, tvec_v.at[pl.ds(0, NUM_CAMS)])

    def build(g, carry):
        base = g * L
        cam = base + iota
        x = plsc.load_gather(rvec_v, [cam, _full(0)])
        y = plsc.load_gather(rvec_v, [cam, _full(1)])
        z = plsc.load_gather(rvec_v, [cam, _full(2)])
        w = plsc.load_gather(rvec_v, [cam, _full(3)])
        t0 = plsc.load_gather(tvec_v, [cam, _full(0)])
        t1 = plsc.load_gather(tvec_v, [cam, _full(1)])
        t2 = plsc.load_gather(tvec_v, [cam, _full(2)])
        theta2 = 1e-5 + x * x + y * y + z * z + w * w
        a = 2.0 / theta2
        axx = a * x * x
        ayy = a * y * y
        azz = a * z * z
        axy = a * x * y
        axz = a * x * z
        ayz = a * y * z
        axw = a * x * w
        ayw = a * y * w
        azw = a * z * w
        r00 = 1.0 - ayy - azz
        r01 = axy - azw
        r02 = axz + ayw
        r10 = axy + azw
        r11 = 1.0 - axx - azz
        r12 = ayz - axw
        r20 = axz - ayw
        r21 = ayz + axw
        r22 = 1.0 - axx - ayy

        def packw(ea, eb):
            return plsc.bitcast(
                plsc.pack(ea, eb, format=plsc.PackFormat.INTERLEAVED),
                jnp.float32)

        tbl_v[0, pl.ds(base, L)] = packw(r00, r01)
        tbl_v[1, pl.ds(base, L)] = packw(r02, t0)
        tbl_v[2, pl.ds(base, L)] = packw(r10, r11)
        tbl_v[3, pl.ds(base, L)] = packw(r12, t1)
        tbl_v[4, pl.ds(base, L)] = packw(r20, r21)
        tbl_v[5, pl.ds(base, L)] = packw(r22, t2)
        return carry

    lax.fori_loop(0, CAM_PAD // L, build, 0)

    wbase = wid * RAYS_PER_W
    zeros = jnp.zeros((L,), jnp.float32)

    def in_copies(c):
        p = c % 2
        base = wbase + c * CHUNK
        bblk = base // BLK
        return (
            pltpu.make_async_copy(rays_o_hbm.at[pl.ds(bblk, CBLK)],
                                  o_v[p], sem_in[p]),
            pltpu.make_async_copy(rays_d_hbm.at[pl.ds(bblk, CBLK)],
                                  d_v[p], sem_in[p]),
            pltpu.make_async_copy(ids_hbm.at[pl.ds(base, CHUNK)],
                                  ids_v[p], sem_in[p]),
        )

    def out_copies(c):
        p = c % 2
        bblk = (wbase + c * CHUNK) // BLK
        return (
            pltpu.make_async_copy(oo_v[p], out_o_hbm.at[pl.ds(bblk, CBLK)],
                                  sem_out[p]),
            pltpu.make_async_copy(dd_v[p], out_d_hbm.at[pl.ds(bblk, CBLK)],
                                  sem_out[p]),
        )

    def compute_chunk(c):
        p = c % 2
        o_v_, d_v_, ids_v_, oo_v_, dd_v_ = (
            o_v[p], d_v[p], ids_v[p], oo_v[p], dd_v[p])

        def block_body(b, carry2):
            for j in range(BLK // L):
                l0 = j * L
                ids16 = ids_v_[pl.ds(b * BLK + l0, L)]
                o0 = o_v_[b, 0, pl.ds(l0, L)]
                o1 = o_v_[b, 1, pl.ds(l0, L)]
                o2 = o_v_[b, 2, pl.ds(l0, L)]
                d0 = d_v_[b, 0, pl.ds(l0, L)]
                d1 = d_v_[b, 1, pl.ds(l0, L)]
                d2 = d_v_[b, 2, pl.ds(l0, L)]
                def unpackw(k):
                    w = plsc.load_gather(tbl_v, [_full(k), ids16])
                    return plsc.unpack(plsc.bitcast(w, jnp.bfloat16),
                                       format=plsc.PackFormat.INTERLEAVED)

                c00, c01 = unpackw(0)
                c02, t0 = unpackw(1)
                c10, c11 = unpackw(2)
                c12, t1 = unpackw(3)
                c20, c21 = unpackw(4)
                c22, t2 = unpackw(5)
                oo_v_[b, 0, pl.ds(l0, L)] = o0 + t0
                oo_v_[b, 1, pl.ds(l0, L)] = o1 + t1
                oo_v_[b, 2, pl.ds(l0, L)] = o2 + t2
                oo_v_[b, 3, pl.ds(l0, L)] = zeros
                dd_v_[b, 0, pl.ds(l0, L)] = d0 * c00 + d1 * c01 + d2 * c02
                dd_v_[b, 1, pl.ds(l0, L)] = d0 * c10 + d1 * c11 + d2 * c12
                dd_v_[b, 2, pl.ds(l0, L)] = d0 * c20 + d1 * c21 + d2 * c22
                dd_v_[b, 3, pl.ds(l0, L)] = zeros
            return carry2

        lax.fori_loop(0, CBLK, block_body, 0)

    for cp in in_copies(0):
        cp.start()
    for c in range(NCHUNKS):
        if c + 1 < NCHUNKS:
            for cp in in_copies(c + 1):
                cp.start()
        for cp in in_copies(c):
            cp.wait()
        if c >= 2:
            for cp in out_copies(c - 2):
                cp.wait()
        compute_chunk(c)
        for cp in out_copies(c):
            cp.start()
    for cp in out_copies(NCHUNKS - 2):
        cp.wait()
    for cp in out_copies(NCHUNKS - 1):
        cp.wait()


_sc_kernel = pl.kernel(
    _body,
    out_type=(jax.ShapeDtypeStruct((NBLK, 4, BLK), jnp.float32),
              jax.ShapeDtypeStruct((NBLK, 4, BLK), jnp.float32)),
    mesh=plsc.VectorSubcoreMesh(core_axis_name="c", subcore_axis_name="s"),
    compiler_params=pltpu.CompilerParams(needs_layout_passes=False,
                                         use_tc_tiling_on_sc=False),
    scratch_types=[
        pltpu.VMEM((CAM_PAD, 4), jnp.float32),
        pltpu.VMEM((CAM_PAD, 3), jnp.float32),
        pltpu.VMEM((6, CAM_PAD), jnp.float32),
        pltpu.VMEM((CBLK, 4, BLK), jnp.float32),
        pltpu.VMEM((CBLK, 4, BLK), jnp.float32),
        pltpu.VMEM((CBLK, 4, BLK), jnp.float32),
        pltpu.VMEM((CBLK, 4, BLK), jnp.float32),
        pltpu.VMEM((CHUNK,), jnp.int32),
        pltpu.VMEM((CHUNK,), jnp.int32),
        pltpu.VMEM((CBLK, 4, BLK), jnp.float32),
        pltpu.VMEM((CBLK, 4, BLK), jnp.float32),
        pltpu.VMEM((CBLK, 4, BLK), jnp.float32),
        pltpu.VMEM((CBLK, 4, BLK), jnp.float32),
        pltpu.SemaphoreType.DMA,
        pltpu.SemaphoreType.DMA,
        pltpu.SemaphoreType.DMA,
        pltpu.SemaphoreType.DMA,
    ],
)


def kernel(rays_o, rays_d, rays_id, rvec, tvec):
    ids = rays_id.astype(jnp.int32)
    o3 = rays_o.reshape(NBLK, BLK, 4).transpose(0, 2, 1)
    d3 = rays_d.reshape(NBLK, BLK, 4).transpose(0, 2, 1)
    oo3, dd3 = _sc_kernel(o3, d3, ids, rvec, tvec)
    out_o = oo3.transpose(0, 2, 1).reshape(N_RAYS, 4)[:, :3]
    out_d = dd3.transpose(0, 2, 1).reshape(N_RAYS, 4)[:, :3]
    return (out_o, out_d)


# speedup vs baseline: 52.2510x; 1.0168x over previous
import jax
import jax.numpy as jnp
from jax import lax
from jax.experimental import pallas as pl
from jax.experimental.pallas import tpu as pltpu
from jax.experimental.pallas import tpu_sc as plsc

N_RAYS = 1048576
NUM_CAMS = 1000
CAM_PAD = 1024
NC = 2
NS = 16
L = 16
NW = NC * NS
RAYS_PER_W = N_RAYS // NW
BLK = 128
NBLK = N_RAYS // BLK
CHUNK = 2048
CBLK = CHUNK // BLK
NCHUNKS = RAYS_PER_W // CHUNK


def _full(v):
    return jnp.full((L,), v, dtype=jnp.int32)


def _body(rays_o_hbm, rays_d_hbm, ids_hbm, rvec_hbm, tvec_hbm,
          out_o_hbm, out_d_hbm,
          rvec_v, tvec_v, tbl_v,
          o_v0, o_v1, d_v0, d_v1, ids_v0, ids_v1,
          oo_v0, oo_v1, dd_v0, dd_v1,
          sem_in0, sem_in1, sem_out0, sem_out1):
    wid = lax.axis_index("s") * NC + lax.axis_index("c")
    iota = lax.iota(jnp.int32, L)
    o_v = (o_v0, o_v1)
    d_v = (d_v0, d_v1)
    ids_v = (ids_v0, ids_v1)
    oo_v = (oo_v0, oo_v1)
    dd_v = (dd_v0, dd_v1)
    sem_in = (sem_in0, sem_in1)
    sem_out = (sem_out0, sem_out1)

    pltpu.sync_copy(rvec_hbm, rvec_v.at[pl.ds(0, NUM_CAMS)])
    pltpu.sync_copy(tvec_hbm, tvec_v.at[pl.ds(0, NUM_CAMS)])

    def build(g, carry):
        base = g * L
        cam = base + iota
        x = plsc.load_gather(rvec_v, [cam, _full(0)])
        y = plsc.load_gather(rvec_v, [cam, _full(1)])
        z = plsc.load_gather(rvec_v, [cam, _full(2)])
        w = plsc.load_gather(rvec_v, [cam, _full(3)])
        t0 = plsc.load_gather(tvec_v, [cam, _full(0)])
        t1 = plsc.load_gather(tvec_v, [cam, _full(1)])
        t2 = plsc.load_gather(tvec_v, [cam, _full(2)])
        theta2 = 1e-5 + x * x + y * y + z * z + w * w
        a = 2.0 / theta2
        axx = a * x * x
        ayy = a * y * y
        azz = a * z * z
        axy = a * x * y
        axz = a * x * z
        ayz = a * y * z
        axw = a * x * w
        ayw = a * y * w
        azw = a * z * w
        r00 = 1.0 - ayy - azz
        r01 = axy - azw
        r02 = axz + ayw
        r10 = axy + azw
        r11 = 1.0 - axx - azz
        r12 = ayz - axw
        r20 = axz - ayw
        r21 = ayz + axw
        r22 = 1.0 - axx - ayy

        def packw(ea, eb):
            return plsc.bitcast(
                plsc.pack(ea, eb, format=plsc.PackFormat.INTERLEAVED),
                jnp.float32)

        tbl_v[0, pl.ds(base, L)] = packw(r00, r01)
        tbl_v[1, pl.ds(base, L)] = packw(r02, t0)
        tbl_v[2, pl.ds(base, L)] = packw(r10, r11)
        tbl_v[3, pl.ds(base, L)] = packw(r12, t1)
        tbl_v[4, pl.ds(base, L)] = packw(r20, r21)
        tbl_v[5, pl.ds(base, L)] = packw(r22, t2)
        return carry

    lax.fori_loop(0, CAM_PAD // L, build, 0)

    wbase = wid * RAYS_PER_W

    def in_copies(c):
        p = c % 2
        base = wbase + c * CHUNK
        bblk = base // BLK
        return (
            pltpu.make_async_copy(
                rays_o_hbm.at[pl.ds(bblk, CBLK), pl.ds(0, 3)],
                o_v[p], sem_in[p]),
            pltpu.make_async_copy(
                rays_d_hbm.at[pl.ds(bblk, CBLK), pl.ds(0, 3)],
                d_v[p], sem_in[p]),
            pltpu.make_async_copy(ids_hbm.at[pl.ds(base, CHUNK)],
                                  ids_v[p], sem_in[p]),
        )

    def out_copies(c):
        p = c % 2
        bblk = (wbase + c * CHUNK) // BLK
        return (
            pltpu.make_async_copy(
                oo_v[p], out_o_hbm.at[pl.ds(bblk, CBLK), pl.ds(0, 3)],
                sem_out[p]),
            pltpu.make_async_copy(
                dd_v[p], out_d_hbm.at[pl.ds(bblk, CBLK), pl.ds(0, 3)],
                sem_out[p]),
        )

    def compute_chunk(c):
        p = c % 2
        o_v_, d_v_, ids_v_, oo_v_, dd_v_ = (
            o_v[p], d_v[p], ids_v[p], oo_v[p], dd_v[p])

        def block_body(b, carry2):
            for j in range(BLK // L):
                l0 = j * L
                ids16 = ids_v_[pl.ds(b * BLK + l0, L)]
                o0 = o_v_[b, 0, pl.ds(l0, L)]
                o1 = o_v_[b, 1, pl.ds(l0, L)]
                o2 = o_v_[b, 2, pl.ds(l0, L)]
                d0 = d_v_[b, 0, pl.ds(l0, L)]
                d1 = d_v_[b, 1, pl.ds(l0, L)]
                d2 = d_v_[b, 2, pl.ds(l0, L)]

                def unpackw(k, ids16=ids16):
                    w = plsc.load_gather(tbl_v, [_full(k), ids16])
                    return plsc.unpack(plsc.bitcast(w, jnp.bfloat16),
                                       format=plsc.PackFormat.INTERLEAVED)

                c00, c01 = unpackw(0)
                c02, t0 = unpackw(1)
                c10, c11 = unpackw(2)
                c12, t1 = unpackw(3)
                c20, c21 = unpackw(4)
                c22, t2 = unpackw(5)
                oo_v_[b, 0, pl.ds(l0, L)] = o0 + t0
                oo_v_[b, 1, pl.ds(l0, L)] = o1 + t1
                oo_v_[b, 2, pl.ds(l0, L)] = o2 + t2
                dd_v_[b, 0, pl.ds(l0, L)] = d0 * c00 + d1 * c01 + d2 * c02
                dd_v_[b, 1, pl.ds(l0, L)] = d0 * c10 + d1 * c11 + d2 * c12
                dd_v_[b, 2, pl.ds(l0, L)] = d0 * c20 + d1 * c21 + d2 * c22
            return carry2

        lax.fori_loop(0, CBLK, block_body, 0)

    for cp in in_copies(0):
        cp.start()
    for c in range(NCHUNKS):
        if c + 1 < NCHUNKS:
            for cp in in_copies(c + 1):
                cp.start()
        for cp in in_copies(c):
            cp.wait()
        if c >= 2:
            for cp in out_copies(c - 2):
                cp.wait()
        compute_chunk(c)
        for cp in out_copies(c):
            cp.start()
    for cp in out_copies(NCHUNKS - 2):
        cp.wait()
    for cp in out_copies(NCHUNKS - 1):
        cp.wait()


_sc_kernel = pl.kernel(
    _body,
    out_type=(jax.ShapeDtypeStruct((NBLK, 4, BLK), jnp.float32),
              jax.ShapeDtypeStruct((NBLK, 4, BLK), jnp.float32)),
    mesh=plsc.VectorSubcoreMesh(core_axis_name="c", subcore_axis_name="s"),
    compiler_params=pltpu.CompilerParams(needs_layout_passes=False,
                                         use_tc_tiling_on_sc=False),
    scratch_types=[
        pltpu.VMEM((CAM_PAD, 4), jnp.float32),
        pltpu.VMEM((CAM_PAD, 3), jnp.float32),
        pltpu.VMEM((6, CAM_PAD), jnp.float32),
        pltpu.VMEM((CBLK, 3, BLK), jnp.float32),
        pltpu.VMEM((CBLK, 3, BLK), jnp.float32),
        pltpu.VMEM((CBLK, 3, BLK), jnp.float32),
        pltpu.VMEM((CBLK, 3, BLK), jnp.float32),
        pltpu.VMEM((CHUNK,), jnp.int32),
        pltpu.VMEM((CHUNK,), jnp.int32),
        pltpu.VMEM((CBLK, 3, BLK), jnp.float32),
        pltpu.VMEM((CBLK, 3, BLK), jnp.float32),
        pltpu.VMEM((CBLK, 3, BLK), jnp.float32),
        pltpu.VMEM((CBLK, 3, BLK), jnp.float32),
        pltpu.SemaphoreType.DMA,
        pltpu.SemaphoreType.DMA,
        pltpu.SemaphoreType.DMA,
        pltpu.SemaphoreType.DMA,
    ],
)


def kernel(rays_o, rays_d, rays_id, rvec, tvec):
    ids = rays_id.astype(jnp.int32)
    o3 = rays_o.reshape(NBLK, BLK, 4).transpose(0, 2, 1)
    d3 = rays_d.reshape(NBLK, BLK, 4).transpose(0, 2, 1)
    oo3, dd3 = _sc_kernel(o3, d3, ids, rvec, tvec)
    out_o = oo3.transpose(0, 2, 1).reshape(N_RAYS, 4)[:, :3]
    out_d = dd3.transpose(0, 2, 1).reshape(N_RAYS, 4)[:, :3]
    return (out_o, out_d)


# speedup vs baseline: 52.2773x; 1.0005x over previous
import jax
import jax.numpy as jnp
from jax import lax
from jax.experimental import pallas as pl
from jax.experimental.pallas import tpu as pltpu
from jax.experimental.pallas import tpu_sc as plsc

N_RAYS = 1048576
NUM_CAMS = 1000
CAM_PAD = 1024
NC = 2
NS = 16
L = 16
NW = NC * NS
RAYS_PER_W = N_RAYS // NW
BLK = 128
NBLK = N_RAYS // BLK
CHUNK = 2048
CBLK = CHUNK // BLK
NCHUNKS = RAYS_PER_W // CHUNK


def _full(v):
    return jnp.full((L,), v, dtype=jnp.int32)


def _body(rays_o_hbm, rays_d_hbm, ids_hbm, rvec_hbm, tvec_hbm,
          out_o_hbm, out_d_hbm,
          rvec_v, tvec_v, tbl_v,
          o_v0, o_v1, d_v0, d_v1, ids_v0, ids_v1,
          oo_v0, oo_v1, dd_v0, dd_v1,
          sem_in0, sem_in1, sem_out0, sem_out1):
    wid = lax.axis_index("s") * NC + lax.axis_index("c")
    iota = lax.iota(jnp.int32, L)
    o_v = (o_v0, o_v1)
    d_v = (d_v0, d_v1)
    ids_v = (ids_v0, ids_v1)
    oo_v = (oo_v0, oo_v1)
    dd_v = (dd_v0, dd_v1)
    sem_in = (sem_in0, sem_in1)
    sem_out = (sem_out0, sem_out1)

    pltpu.sync_copy(rvec_hbm, rvec_v.at[pl.ds(0, NUM_CAMS)])
    pltpu.sync_copy(tvec_hbm, tvec_v.at[pl.ds(0, NUM_CAMS)])

    def build(g, carry):
        base = g * L
        cam = base + iota
        x = plsc.load_gather(rvec_v, [cam, _full(0)])
        y = plsc.load_gather(rvec_v, [cam, _full(1)])
        z = plsc.load_gather(rvec_v, [cam, _full(2)])
        w = plsc.load_gather(rvec_v, [cam, _full(3)])
        t0 = plsc.load_gather(tvec_v, [cam, _full(0)])
        t1 = plsc.load_gather(tvec_v, [cam, _full(1)])
        t2 = plsc.load_gather(tvec_v, [cam, _full(2)])
        theta2 = 1e-5 + x * x + y * y + z * z + w * w
        a = 2.0 / theta2
        axx = a * x * x
        ayy = a * y * y
        azz = a * z * z
        axy = a * x * y
        axz = a * x * z
        ayz = a * y * z
        axw = a * x * w
        ayw = a * y * w
        azw = a * z * w
        r00 = 1.0 - ayy - azz
        r01 = axy - azw
        r02 = axz + ayw
        r10 = axy + azw
        r11 = 1.0 - axx - azz
        r12 = ayz - axw
        r20 = axz - ayw
        r21 = ayz + axw
        r22 = 1.0 - axx - ayy

        def packw(ea, eb):
            return plsc.bitcast(
                plsc.pack(ea, eb, format=plsc.PackFormat.INTERLEAVED),
                jnp.float32)

        tbl_v[0, pl.ds(base, L)] = packw(r00, r01)
        tbl_v[1, pl.ds(base, L)] = packw(r02, t0)
        tbl_v[2, pl.ds(base, L)] = packw(r10, r11)
        tbl_v[3, pl.ds(base, L)] = packw(r12, t1)
        tbl_v[4, pl.ds(base, L)] = packw(r20, r21)
        tbl_v[5, pl.ds(base, L)] = packw(r22, t2)
        return carry

    lax.fori_loop(0, CAM_PAD // L, build, 0)

    wbase = wid * RAYS_PER_W

    def in_copies(c):
        p = c % 2
        base = wbase + c * CHUNK
        bblk = base // BLK
        return (
            pltpu.make_async_copy(
                rays_o_hbm.at[pl.ds(bblk, CBLK), pl.ds(0, 3)],
                o_v[p], sem_in[p]),
            pltpu.make_async_copy(
                rays_d_hbm.at[pl.ds(bblk, CBLK), pl.ds(0, 3)],
                d_v[p], sem_in[p]),
            pltpu.make_async_copy(ids_hbm.at[pl.ds(base, CHUNK)],
                                  ids_v[p], sem_in[p]),
        )

    def out_copies(c):
        p = c % 2
        bblk = (wbase + c * CHUNK) // BLK
        return (
            pltpu.make_async_copy(
                oo_v[p], out_o_hbm.at[pl.ds(bblk, CBLK), pl.ds(0, 3)],
                sem_out[p]),
            pltpu.make_async_copy(
                dd_v[p], out_d_hbm.at[pl.ds(bblk, CBLK), pl.ds(0, 3)],
                sem_out[p]),
        )

    def compute_chunk(c):
        p = c % 2
        o_v_, d_v_, ids_v_, oo_v_, dd_v_ = (
            o_v[p], d_v[p], ids_v[p], oo_v[p], dd_v[p])

        def block_body(b, carry2):
            for j in range(BLK // L):
                l0 = j * L
                ids16 = ids_v_[pl.ds(b * BLK + l0, L)]
                o0 = o_v_[b, 0, pl.ds(l0, L)]
                o1 = o_v_[b, 1, pl.ds(l0, L)]
                o2 = o_v_[b, 2, pl.ds(l0, L)]
                d0 = d_v_[b, 0, pl.ds(l0, L)]
                d1 = d_v_[b, 1, pl.ds(l0, L)]
                d2 = d_v_[b, 2, pl.ds(l0, L)]

                def unpackw(k, ids16=ids16):
                    w = plsc.load_gather(tbl_v, [_full(k), ids16])
                    return plsc.unpack(plsc.bitcast(w, jnp.bfloat16),
                                       format=plsc.PackFormat.INTERLEAVED)

                c00, c01 = unpackw(0)
                c02, t0 = unpackw(1)
                c10, c11 = unpackw(2)
                c12, t1 = unpackw(3)
                c20, c21 = unpackw(4)
                c22, t2 = unpackw(5)
                oo_v_[b, 0, pl.ds(l0, L)] = o0 + t0
                oo_v_[b, 1, pl.ds(l0, L)] = o1 + t1
                oo_v_[b, 2, pl.ds(l0, L)] = o2 + t2
                dd_v_[b, 0, pl.ds(l0, L)] = d0 * c00 + d1 * c01 + d2 * c02
                dd_v_[b, 1, pl.ds(l0, L)] = d0 * c10 + d1 * c11 + d2 * c12
                dd_v_[b, 2, pl.ds(l0, L)] = d0 * c20 + d1 * c21 + d2 * c22
            return carry2

        lax.fori_loop(0, CBLK, block_body, 0)

    for cp in in_copies(0):
        cp.start()
    for c in range(NCHUNKS):
        if c + 1 < NCHUNKS:
            for cp in in_copies(c + 1):
                cp.start()
        for cp in in_copies(c):
            cp.wait()
        if c >= 2:
            for cp in out_copies(c - 2):
                cp.wait()
        compute_chunk(c)
        for cp in out_copies(c):
            cp.start()
    for cp in out_copies(NCHUNKS - 2):
        cp.wait()
    for cp in out_copies(NCHUNKS - 1):
        cp.wait()


_sc_kernel = pl.kernel(
    _body,
    out_type=(jax.ShapeDtypeStruct((NBLK, 4, BLK), jnp.float32),
              jax.ShapeDtypeStruct((NBLK, 4, BLK), jnp.float32)),
    mesh=plsc.VectorSubcoreMesh(core_axis_name="c", subcore_axis_name="s"),
    compiler_params=pltpu.CompilerParams(needs_layout_passes=False,
                                         use_tc_tiling_on_sc=False,
                                         skip_device_barrier=True),
    scratch_types=[
        pltpu.VMEM((CAM_PAD, 4), jnp.float32),
        pltpu.VMEM((CAM_PAD, 3), jnp.float32),
        pltpu.VMEM((6, CAM_PAD), jnp.float32),
        pltpu.VMEM((CBLK, 3, BLK), jnp.float32),
        pltpu.VMEM((CBLK, 3, BLK), jnp.float32),
        pltpu.VMEM((CBLK, 3, BLK), jnp.float32),
        pltpu.VMEM((CBLK, 3, BLK), jnp.float32),
        pltpu.VMEM((CHUNK,), jnp.int32),
        pltpu.VMEM((CHUNK,), jnp.int32),
        pltpu.VMEM((CBLK, 3, BLK), jnp.float32),
        pltpu.VMEM((CBLK, 3, BLK), jnp.float32),
        pltpu.VMEM((CBLK, 3, BLK), jnp.float32),
        pltpu.VMEM((CBLK, 3, BLK), jnp.float32),
        pltpu.SemaphoreType.DMA,
        pltpu.SemaphoreType.DMA,
        pltpu.SemaphoreType.DMA,
        pltpu.SemaphoreType.DMA,
    ],
)


def kernel(rays_o, rays_d, rays_id, rvec, tvec):
    ids = rays_id.astype(jnp.int32)
    o3 = rays_o.reshape(NBLK, BLK, 4).transpose(0, 2, 1)
    d3 = rays_d.reshape(NBLK, BLK, 4).transpose(0, 2, 1)
    oo3, dd3 = _sc_kernel(o3, d3, ids, rvec, tvec)
    out_o = oo3.transpose(0, 2, 1).reshape(N_RAYS, 4)[:, :3]
    out_d = dd3.transpose(0, 2, 1).reshape(N_RAYS, 4)[:, :3]
    return (out_o, out_d)


# speedup vs baseline: 54.1521x; 1.0359x over previous
import jax
import jax.numpy as jnp
from jax import lax
from jax.experimental import pallas as pl
from jax.experimental.pallas import tpu as pltpu
from jax.experimental.pallas import tpu_sc as plsc

N_RAYS = 1048576
NUM_CAMS = 1000
CAM_PAD = 1024
NC = 2
NS = 16
L = 16
NW = NC * NS
RAYS_PER_W = N_RAYS // NW
BLK = 128
NBLK = N_RAYS // BLK
CHUNK = 4096
CBLK = CHUNK // BLK
NCHUNKS = RAYS_PER_W // CHUNK


def _full(v):
    return jnp.full((L,), v, dtype=jnp.int32)


def _body(rays_o_hbm, rays_d_hbm, ids_hbm, rvec_hbm, tvec_hbm,
          out_o_hbm, out_d_hbm,
          rvec_v, tvec_v, tbl_v,
          o_v0, o_v1, d_v0, d_v1, ids_v0, ids_v1,
          oo_v0, oo_v1, dd_v0, dd_v1,
          sem_in0, sem_in1, sem_out0, sem_out1):
    wid = lax.axis_index("s") * NC + lax.axis_index("c")
    iota = lax.iota(jnp.int32, L)
    o_v = (o_v0, o_v1)
    d_v = (d_v0, d_v1)
    ids_v = (ids_v0, ids_v1)
    oo_v = (oo_v0, oo_v1)
    dd_v = (dd_v0, dd_v1)
    sem_in = (sem_in0, sem_in1)
    sem_out = (sem_out0, sem_out1)

    pltpu.sync_copy(rvec_hbm, rvec_v.at[pl.ds(0, NUM_CAMS)])
    pltpu.sync_copy(tvec_hbm, tvec_v.at[pl.ds(0, NUM_CAMS)])

    def build(g, carry):
        base = g * L
        cam = base + iota
        x = plsc.load_gather(rvec_v, [cam, _full(0)])
        y = plsc.load_gather(rvec_v, [cam, _full(1)])
        z = plsc.load_gather(rvec_v, [cam, _full(2)])
        w = plsc.load_gather(rvec_v, [cam, _full(3)])
        t0 = plsc.load_gather(tvec_v, [cam, _full(0)])
        t1 = plsc.load_gather(tvec_v, [cam, _full(1)])
        t2 = plsc.load_gather(tvec_v, [cam, _full(2)])
        theta2 = 1e-5 + x * x + y * y + z * z + w * w
        a = 2.0 / theta2
        axx = a * x * x
        ayy = a * y * y
        azz = a * z * z
        axy = a * x * y
        axz = a * x * z
        ayz = a * y * z
        axw = a * x * w
        ayw = a * y * w
        azw = a * z * w
        r00 = 1.0 - ayy - azz
        r01 = axy - azw
        r02 = axz + ayw
        r10 = axy + azw
        r11 = 1.0 - axx - azz
        r12 = ayz - axw
        r20 = axz - ayw
        r21 = ayz + axw
        r22 = 1.0 - axx - ayy

        def packw(ea, eb):
            return plsc.bitcast(
                plsc.pack(ea, eb, format=plsc.PackFormat.INTERLEAVED),
                jnp.float32)

        tbl_v[0, pl.ds(base, L)] = packw(r00, r01)
        tbl_v[1, pl.ds(base, L)] = packw(r02, t0)
        tbl_v[2, pl.ds(base, L)] = packw(r10, r11)
        tbl_v[3, pl.ds(base, L)] = packw(r12, t1)
        tbl_v[4, pl.ds(base, L)] = packw(r20, r21)
        tbl_v[5, pl.ds(base, L)] = packw(r22, t2)
        return carry

    lax.fori_loop(0, CAM_PAD // L, build, 0)

    wbase = wid * RAYS_PER_W

    def in_copies(c):
        p = c % 2
        base = wbase + c * CHUNK
        bblk = base // BLK
        return (
            pltpu.make_async_copy(
                rays_o_hbm.at[pl.ds(bblk, CBLK), pl.ds(0, 3)],
                o_v[p], sem_in[p]),
            pltpu.make_async_copy(
                rays_d_hbm.at[pl.ds(bblk, CBLK), pl.ds(0, 3)],
                d_v[p], sem_in[p]),
            pltpu.make_async_copy(ids_hbm.at[pl.ds(base, CHUNK)],
                                  ids_v[p], sem_in[p]),
        )

    def out_copies(c):
        p = c % 2
        bblk = (wbase + c * CHUNK) // BLK
        return (
            pltpu.make_async_copy(
                oo_v[p], out_o_hbm.at[pl.ds(bblk, CBLK), pl.ds(0, 3)],
                sem_out[p]),
            pltpu.make_async_copy(
                dd_v[p], out_d_hbm.at[pl.ds(bblk, CBLK), pl.ds(0, 3)],
                sem_out[p]),
        )

    def compute_chunk(c):
        p = c % 2
        o_v_, d_v_, ids_v_, oo_v_, dd_v_ = (
            o_v[p], d_v[p], ids_v[p], oo_v[p], dd_v[p])

        def block_body(b, carry2):
            for j in range(BLK // L):
                l0 = j * L
                ids16 = ids_v_[pl.ds(b * BLK + l0, L)]
                o0 = o_v_[b, 0, pl.ds(l0, L)]
                o1 = o_v_[b, 1, pl.ds(l0, L)]
                o2 = o_v_[b, 2, pl.ds(l0, L)]
                d0 = d_v_[b, 0, pl.ds(l0, L)]
                d1 = d_v_[b, 1, pl.ds(l0, L)]
                d2 = d_v_[b, 2, pl.ds(l0, L)]

                def unpackw(k, ids16=ids16):
                    w = plsc.load_gather(tbl_v, [_full(k), ids16])
                    return plsc.unpack(plsc.bitcast(w, jnp.bfloat16),
                                       format=plsc.PackFormat.INTERLEAVED)

                c00, c01 = unpackw(0)
                c02, t0 = unpackw(1)
                c10, c11 = unpackw(2)
                c12, t1 = unpackw(3)
                c20, c21 = unpackw(4)
                c22, t2 = unpackw(5)
                oo_v_[b, 0, pl.ds(l0, L)] = o0 + t0
                oo_v_[b, 1, pl.ds(l0, L)] = o1 + t1
                oo_v_[b, 2, pl.ds(l0, L)] = o2 + t2
                dd_v_[b, 0, pl.ds(l0, L)] = d0 * c00 + d1 * c01 + d2 * c02
                dd_v_[b, 1, pl.ds(l0, L)] = d0 * c10 + d1 * c11 + d2 * c12
                dd_v_[b, 2, pl.ds(l0, L)] = d0 * c20 + d1 * c21 + d2 * c22
            return carry2

        lax.fori_loop(0, CBLK, block_body, 0)

    for cp in in_copies(0):
        cp.start()
    for c in range(NCHUNKS):
        if c + 1 < NCHUNKS:
            for cp in in_copies(c + 1):
                cp.start()
        for cp in in_copies(c):
            cp.wait()
        if c >= 2:
            for cp in out_copies(c - 2):
                cp.wait()
        compute_chunk(c)
        for cp in out_copies(c):
            cp.start()
    for cp in out_copies(NCHUNKS - 2):
        cp.wait()
    for cp in out_copies(NCHUNKS - 1):
        cp.wait()


_sc_kernel = pl.kernel(
    _body,
    out_type=(jax.ShapeDtypeStruct((NBLK, 4, BLK), jnp.float32),
              jax.ShapeDtypeStruct((NBLK, 4, BLK), jnp.float32)),
    mesh=plsc.VectorSubcoreMesh(core_axis_name="c", subcore_axis_name="s"),
    compiler_params=pltpu.CompilerParams(needs_layout_passes=False,
                                         use_tc_tiling_on_sc=False),
    scratch_types=[
        pltpu.VMEM((CAM_PAD, 4), jnp.float32),
        pltpu.VMEM((CAM_PAD, 3), jnp.float32),
        pltpu.VMEM((6, CAM_PAD), jnp.float32),
        pltpu.VMEM((CBLK, 3, BLK), jnp.float32),
        pltpu.VMEM((CBLK, 3, BLK), jnp.float32),
        pltpu.VMEM((CBLK, 3, BLK), jnp.float32),
        pltpu.VMEM((CBLK, 3, BLK), jnp.float32),
        pltpu.VMEM((CHUNK,), jnp.int32),
        pltpu.VMEM((CHUNK,), jnp.int32),
        pltpu.VMEM((CBLK, 3, BLK), jnp.float32),
        pltpu.VMEM((CBLK, 3, BLK), jnp.float32),
        pltpu.VMEM((CBLK, 3, BLK), jnp.float32),
        pltpu.VMEM((CBLK, 3, BLK), jnp.float32),
        pltpu.SemaphoreType.DMA,
        pltpu.SemaphoreType.DMA,
        pltpu.SemaphoreType.DMA,
        pltpu.SemaphoreType.DMA,
    ],
)


def kernel(rays_o, rays_d, rays_id, rvec, tvec):
    ids = rays_id.astype(jnp.int32)
    o3 = rays_o.reshape(NBLK, BLK, 4).transpose(0, 2, 1)
    d3 = rays_d.reshape(NBLK, BLK, 4).transpose(0, 2, 1)
    oo3, dd3 = _sc_kernel(o3, d3, ids, rvec, tvec)
    out_o = oo3.transpose(0, 2, 1).reshape(N_RAYS, 4)[:, :3]
    out_d = dd3.transpose(0, 2, 1).reshape(N_RAYS, 4)[:, :3]
    return (out_o, out_d)
